# baseline (speedup 1.0000x reference)
import functools

import jax
import jax.numpy as jnp
from jax import lax
from jax.experimental import pallas as pl
from jax.experimental.pallas import tpu as pltpu
from jax.experimental.pallas import tpu_sc as plsc

D = 128
N_NODES = 50000

R_RANGE = 12544
NP_CORE = 2
OUT_ROWS = 4 * R_RANGE
EPT = 12544
E_PAD = 16 * EPT
CHUNK = 128
N_CHUNKS = EPT // CHUNK
ZROWS = 784
ZCH = 112
TRASH = R_RANGE

RC = 8448
NPC_CORE = 3
OUT_ROWS_C = 6 * RC
ZROWS_C = 528
ZCH_C = 88


def _zero_rows(buf, nrows):
    zv16 = jnp.zeros((16,), jnp.float32)

    def body(i, _):
        for k in range(8):
            buf[i, pl.ds(k * 16, 16)] = zv16
        return 0
    lax.fori_loop(0, nrows, body, 0)


def _sc_agg_body(x_hbm, src_hbm, dst_hbm, w_hbm, out_s_hbm,
                 acc, gbuf, srcv, dstv, wv, idxb, sem):
    c = lax.axis_index("c")
    s = lax.axis_index("s")
    row0 = s * ZROWS
    for p_i in range(NP_CORE):
        p = NP_CORE * c + p_i
        lo = p * R_RANGE

        _zero_rows(gbuf, ZCH)
        for j in range(7):
            pltpu.sync_copy(gbuf.at[pl.ds(0, ZCH)],
                            acc.at[pl.ds(row0 + j * ZCH, ZCH)])
        plsc.subcore_barrier()

        def chunk_body(ch, _):
            base = s * EPT + ch * CHUNK
            pltpu.sync_copy(src_hbm.at[pl.ds(base, CHUNK)], srcv)
            pltpu.sync_copy(dst_hbm.at[pl.ds(base, CHUNK)], dstv)
            pltpu.sync_copy(w_hbm.at[pl.ds(base, CHUNK)], wv)
            pltpu.async_copy(x_hbm.at[srcv], gbuf, sem).wait()
            for g in range(CHUNK // 16):
                d16 = dstv[pl.ds(g * 16, 16)]
                loc = d16 - lo
                inr = (loc >= 0) & (loc < R_RANGE)
                idxb[pl.ds(g * 16, 16)] = jnp.where(inr, loc, TRASH)

            def edge_group(g, _):
                w16 = wv[pl.ds(g * 16, 16)]
                for j in range(16):
                    e = g * 16 + j
                    we = w16[j]
                    for k in range(8):
                        sl = pl.ds(k * 16, 16)
                        gbuf[e, sl] = gbuf[e, sl] * we
                return 0
            lax.fori_loop(0, CHUNK // 16, edge_group, 0)
            pltpu.sync_copy(gbuf, acc.at[idxb], add=True)
            return 0
        lax.fori_loop(0, N_CHUNKS, chunk_body, 0)
        plsc.subcore_barrier()

        for j in range(7):
            pltpu.sync_copy(acc.at[pl.ds(row0 + j * ZCH, ZCH)],
                            out_s_hbm.at[pl.ds(lo + row0 + j * ZCH, ZCH)])
        plsc.subcore_barrier()


_sc_agg_call = pl.kernel(
    _sc_agg_body,
    out_type=jax.ShapeDtypeStruct((OUT_ROWS, D), jnp.float32),
    mesh=plsc.VectorSubcoreMesh(core_axis_name="c", subcore_axis_name="s"),
    scratch_types=[
        pltpu.VMEM_SHARED((R_RANGE + 1, D), jnp.float32),
        pltpu.VMEM((CHUNK, D), jnp.float32),
        pltpu.VMEM((CHUNK,), jnp.int32),
        pltpu.VMEM((CHUNK,), jnp.int32),
        pltpu.VMEM((CHUNK,), jnp.float32),
        pltpu.VMEM((CHUNK,), jnp.int32),
        pltpu.SemaphoreType.DMA,
    ],
)


def _sc_cnt_body(dst_hbm, out_c_hbm, acc, cone, zb, dstv, idxb):
    c = lax.axis_index("c")
    s = lax.axis_index("s")

    _zero_rows(zb, ZCH_C)
    fo = jnp.where(lax.iota(jnp.int32, 16) == 0, 1.0, 0.0).astype(jnp.float32)
    zv16 = jnp.zeros((16,), jnp.float32)

    def init_cone(i, _):
        cone[i, pl.ds(0, 16)] = fo
        for k in range(1, 8):
            cone[i, pl.ds(k * 16, 16)] = zv16
        return 0
    lax.fori_loop(0, CHUNK, init_cone, 0)

    row0 = s * ZROWS_C
    for p_i in range(NPC_CORE):
        p = NPC_CORE * c + p_i
        lo = p * RC
        for j in range(6):
            pltpu.sync_copy(zb, acc.at[pl.ds(row0 + j * ZCH_C, ZCH_C)])
        plsc.subcore_barrier()

        def chunk_body(ch, _):
            base = s * EPT + ch * CHUNK
            pltpu.sync_copy(dst_hbm.at[pl.ds(base, CHUNK)], dstv)
            for g in range(CHUNK // 16):
                d16 = dstv[pl.ds(g * 16, 16)]
                loc = d16 - lo
                inr = (loc >= 0) & (loc < RC)
                idxb[pl.ds(g * 16, 16)] = jnp.where(inr, loc, RC)
            pltpu.sync_copy(cone, acc.at[idxb], add=True)
            return 0
        lax.fori_loop(0, N_CHUNKS, chunk_body, 0)
        plsc.subcore_barrier()

        for j in range(6):
            pltpu.sync_copy(acc.at[pl.ds(row0 + j * ZCH_C, ZCH_C)],
                            out_c_hbm.at[pl.ds(lo + row0 + j * ZCH_C, ZCH_C)])
        plsc.subcore_barrier()


_sc_cnt_call = pl.kernel(
    _sc_cnt_body,
    out_type=jax.ShapeDtypeStruct((OUT_ROWS_C, D), jnp.float32),
    mesh=plsc.VectorSubcoreMesh(core_axis_name="c", subcore_axis_name="s"),
    scratch_types=[
        pltpu.VMEM_SHARED((RC + 1, D), jnp.float32),
        pltpu.VMEM((CHUNK, D), jnp.float32),
        pltpu.VMEM((ZCH_C, D), jnp.float32),
        pltpu.VMEM((CHUNK,), jnp.int32),
        pltpu.VMEM((CHUNK,), jnp.int32),
    ],
)


def _pad_edges(src, dst, w):
    pad = E_PAD - src.shape[0]
    srcp = jnp.pad(src, (0, pad))
    dstp = jnp.pad(dst, (0, pad), constant_values=1 << 28)
    wp = jnp.pad(w, (0, pad))
    return srcp, dstp, wp


def _sc_agg(x_src, srcp, dstp, w):
    pad = E_PAD - w.shape[0]
    wp = jnp.pad(w, (0, pad))
    return _sc_agg_call(x_src, srcp, dstp, wp)


def _sc_cnt(dst):
    dstp = jnp.pad(dst, (0, E_PAD - dst.shape[0]), constant_values=1 << 28)
    return _sc_cnt_call(dstp)


def _w_kernel(ea_ref, We_ref, be_ref, out_ref):
    x = ea_ref[0]
    y = jax.nn.relu(
        jnp.dot(x, We_ref[...], preferred_element_type=jnp.float32) + be_ref[...]
    )
    out_ref[0, 0, :] = jnp.mean(y, axis=1)


def _edge_w(ea, We, be):
    E, ed = ea.shape
    B = 1000
    nb = E // B
    ea_p = jnp.zeros((E, 16), jnp.float32).at[:, :ed].set(ea).reshape(nb, B, 16)
    We_p = jnp.zeros((16, D), jnp.float32).at[:ed, :].set(We)
    out = pl.pallas_call(
        _w_kernel,
        grid=(nb,),
        in_specs=[
            pl.BlockSpec((1, B, 16), lambda i: (i, 0, 0)),
            pl.BlockSpec((16, D), lambda i: (0, 0)),
            pl.BlockSpec((1, D), lambda i: (0, 0)),
        ],
        out_specs=pl.BlockSpec((1, 1, B), lambda i: (i, 0, 0)),
        out_shape=jax.ShapeDtypeStruct((nb, 1, B), jnp.float32),
    )(ea_p, We_p, be.reshape(1, D))
    return out.reshape(E)


def _combine2_kernel(acc1_ref, cnt1_ref, acc2_ref, cnt2_ref, h_ref, Wl1_ref,
                     Wl2_ref, Wr_ref, bias_ref, g_ref, b_ref, out_ref, *,
                     residual):
    a1 = acc1_ref[...]
    c1 = cnt1_ref[:, 0:1]
    a2 = acc2_ref[...]
    c2 = cnt2_ref[:, 0:1]
    h = h_ref[...]
    y = jnp.dot(a1 / jnp.maximum(c1, 1.0), Wl1_ref[...],
                preferred_element_type=jnp.float32)
    y = y + jnp.dot(a2 / jnp.maximum(c2, 1.0), Wl2_ref[...],
                    preferred_element_type=jnp.float32)
    y = y + jnp.dot(h, Wr_ref[...], preferred_element_type=jnp.float32)
    y = y + bias_ref[...]
    y = jax.nn.relu(y)
    mu = jnp.mean(y, axis=1, keepdims=True)
    yc = y - mu
    var = jnp.mean(yc * yc, axis=1, keepdims=True)
    out = yc * jax.lax.rsqrt(var + 1e-5) * g_ref[...] + b_ref[...]
    if residual:
        out = out + h
    out_ref[...] = out


def _combine1_kernel(acc1_ref, cnt1_ref, h_ref, Wl1_ref, Wr_ref,
                     bias_ref, g_ref, b_ref, out_ref, *, residual):
    a1 = acc1_ref[...]
    c1 = cnt1_ref[:, 0:1]
    h = h_ref[...]
    y = jnp.dot(a1 / jnp.maximum(c1, 1.0), Wl1_ref[...],
                preferred_element_type=jnp.float32)
    y = y + jnp.dot(h, Wr_ref[...], preferred_element_type=jnp.float32)
    y = y + bias_ref[...]
    y = jax.nn.relu(y)
    mu = jnp.mean(y, axis=1, keepdims=True)
    yc = y - mu
    var = jnp.mean(yc * yc, axis=1, keepdims=True)
    out = yc * jax.lax.rsqrt(var + 1e-5) * g_ref[...] + b_ref[...]
    if residual:
        out = out + h
    out_ref[...] = out


def _combine2(acc1, cnt1, acc2, cnt2, h, Wl1, Wl2, Wr, bias, g, b, residual):
    N = h.shape[0]
    B = 1000
    nb = N // B
    body = functools.partial(_combine2_kernel, residual=residual)
    return pl.pallas_call(
        body,
        grid=(nb,),
        in_specs=[
            pl.BlockSpec((B, D), lambda i: (i, 0)),
            pl.BlockSpec((B, D), lambda i: (i, 0)),
            pl.BlockSpec((B, D), lambda i: (i, 0)),
            pl.BlockSpec((B, D), lambda i: (i, 0)),
            pl.BlockSpec((B, D), lambda i: (i, 0)),
            pl.BlockSpec((D, D), lambda i: (0, 0)),
            pl.BlockSpec((D, D), lambda i: (0, 0)),
            pl.BlockSpec((D, D), lambda i: (0, 0)),
            pl.BlockSpec((1, D), lambda i: (0, 0)),
            pl.BlockSpec((1, D), lambda i: (0, 0)),
            pl.BlockSpec((1, D), lambda i: (0, 0)),
        ],
        out_specs=pl.BlockSpec((B, D), lambda i: (i, 0)),
        out_shape=jax.ShapeDtypeStruct((N, D), jnp.float32),
    )(acc1, cnt1, acc2, cnt2, h, Wl1, Wl2, Wr, bias.reshape(1, D),
      g.reshape(1, D), b.reshape(1, D))


def _combine1(acc1, cnt1, h, Wl1, Wr, bias, g, b, residual):
    N = h.shape[0]
    B = 1000
    nb = N // B
    body = functools.partial(_combine1_kernel, residual=residual)
    return pl.pallas_call(
        body,
        grid=(nb,),
        in_specs=[
            pl.BlockSpec((B, D), lambda i: (i, 0)),
            pl.BlockSpec((B, D), lambda i: (i, 0)),
            pl.BlockSpec((B, D), lambda i: (i, 0)),
            pl.BlockSpec((D, D), lambda i: (0, 0)),
            pl.BlockSpec((D, D), lambda i: (0, 0)),
            pl.BlockSpec((1, D), lambda i: (0, 0)),
            pl.BlockSpec((1, D), lambda i: (0, 0)),
            pl.BlockSpec((1, D), lambda i: (0, 0)),
        ],
        out_specs=pl.BlockSpec((B, D), lambda i: (i, 0)),
        out_shape=jax.ShapeDtypeStruct((N, D), jnp.float32),
    )(acc1, cnt1, h, Wl1, Wr, bias.reshape(1, D), g.reshape(1, D),
      b.reshape(1, D))


def kernel(x_user, x_place, ei_uu, ea_uu, ei_up, ea_up, ei_pu, ea_pu, params):
    h_u, h_p = x_user, x_place
    src_uu, dst_uu, _ = _pad_edges(ei_uu[0], ei_uu[1], ei_uu[1].astype(jnp.float32))
    src_pu, dst_pu, _ = _pad_edges(ei_pu[0], ei_pu[1], ei_pu[1].astype(jnp.float32))
    src_up, dst_up, _ = _pad_edges(ei_up[0], ei_up[1], ei_up[1].astype(jnp.float32))
    c_uu = _sc_cnt(ei_uu[1])[:N_NODES]
    c_pu = _sc_cnt(ei_pu[1])[:N_NODES]
    c_up = _sc_cnt(ei_up[1])[:N_NODES]
    for l in range(2):
        lp = params['layer%d' % l]
        w_uu = _edge_w(ea_uu, lp['uu']['We'], lp['uu']['be'])
        w_pu = _edge_w(ea_pu, lp['pu']['We'], lp['pu']['be'])
        w_up = _edge_w(ea_up, lp['up']['We'], lp['up']['be'])
        s_uu = _sc_agg(h_u, src_uu, dst_uu, w_uu)
        s_pu = _sc_agg(h_p, src_pu, dst_pu, w_pu)
        s_up = _sc_agg(h_u, src_up, dst_up, w_up)
        bias_u = (lp['uu']['bl'] + lp['uu']['br'] + lp['pu']['bl'] +
                  lp['pu']['br'])
        bias_p = lp['up']['bl'] + lp['up']['br']
        new_u = _combine2(s_uu[:N_NODES], c_uu, s_pu[:N_NODES],
                          c_pu, h_u, lp['uu']['Wl'], lp['pu']['Wl'],
                          lp['uu']['Wr'] + lp['pu']['Wr'], bias_u,
                          lp['ln_u_g'], lp['ln_u_b'], residual=(l > 0))
        new_p = _combine1(s_up[:N_NODES], c_up, h_p, lp['up']['Wl'],
                          lp['up']['Wr'], bias_p, lp['ln_p_g'], lp['ln_p_b'],
                          residual=(l > 0))
        h_u, h_p = new_u, new_p
    return h_u, h_p
